# Initial kernel scaffold; baseline (speedup 1.0000x reference)
import jax
import jax.numpy as jnp
from jax.experimental import pallas as pl


def kernel(x, edge_index, W1, b1, W2, b2, fc1_w, fc1_b, fc2_w, fc2_b):
    raise NotImplementedError("write your pallas kernel here")


# speedup vs baseline: 15.4476x; 15.4476x over previous
import functools

import jax
import jax.numpy as jnp
from jax import lax
from jax.experimental import pallas as pl
from jax.experimental.pallas import tpu as pltpu
from jax.experimental.pallas import tpu_sc as plsc

NN = 10000
EE = 320000
DD = 128
NC = 2
NS = 16
NW = NC * NS
CH = 128
NCHUNK = EE // CH
ITERS = -(-NCHUNK // NW)
SROW = 624
ZR = 208
TAIL = NN - NS * SROW
DEGW = 16
RB = 400


def _sc_mesh():
    return plsc.VectorSubcoreMesh(core_axis_name="c", subcore_axis_name="s")


def _zero_acc(zerov, acc, sid):
    @pl.loop(0, SROW // ZR)
    def _(j):
        pltpu.sync_copy(zerov, acc.at[pl.ds(sid * SROW + j * ZR, ZR)])

    @pl.when(sid == NS - 1)
    def _():
        pltpu.sync_copy(zerov.at[pl.ds(0, TAIL)], acc.at[pl.ds(NS * SROW, TAIL)])


def _copy_out(acc, out_hbm, cid, sid):
    @pl.loop(0, SROW // ZR)
    def _(j):
        off = sid * SROW + j * ZR
        pltpu.sync_copy(acc.at[pl.ds(off, ZR)], out_hbm.at[cid, pl.ds(off, ZR)])

    @pl.when(sid == NS - 1)
    def _():
        off = NS * SROW
        pltpu.sync_copy(acc.at[pl.ds(off, TAIL)], out_hbm.at[cid, pl.ds(off, TAIL)])


def _sc_degree(dst_arr):
    @functools.partial(
        pl.kernel,
        out_type=jax.ShapeDtypeStruct((NC, NN, DEGW), jnp.float32),
        mesh=_sc_mesh(),
        scratch_types=[
            pltpu.VMEM((CH,), jnp.int32),
            pltpu.VMEM((CH, DEGW), jnp.float32),
            pltpu.VMEM((ZR, DEGW), jnp.float32),
            pltpu.VMEM_SHARED((NN, DEGW), jnp.float32),
        ],
    )
    def deg_kernel(dst_hbm, out_hbm, dstv, onesv, zerov, acc):
        cid = lax.axis_index("c")
        sid = lax.axis_index("s")
        wid = sid * NC + cid

        @pl.loop(0, CH)
        def _(i):
            onesv[i, :] = jnp.full((DEGW,), 1.0, jnp.float32)

        @pl.loop(0, ZR)
        def _(i):
            zerov[i, :] = jnp.zeros((DEGW,), jnp.float32)

        _zero_acc(zerov, acc, sid)
        plsc.subcore_barrier()

        @pl.loop(0, ITERS)
        def _(it):
            chunk = it * NW + wid

            @pl.when(chunk < NCHUNK)
            def _():
                pltpu.sync_copy(dst_hbm.at[pl.ds(chunk * CH, CH)], dstv)
                pltpu.sync_copy(onesv, acc.at[dstv], add=True)

        plsc.subcore_barrier()
        _copy_out(acc, out_hbm, cid, sid)

    return deg_kernel(dst_arr)


def _sc_scatter(src_arr, dst_arr, y):
    @functools.partial(
        pl.kernel,
        out_type=jax.ShapeDtypeStruct((NC, NN, DD), jnp.float32),
        mesh=_sc_mesh(),
        scratch_types=[
            pltpu.VMEM((CH,), jnp.int32),
            pltpu.VMEM((CH,), jnp.int32),
            pltpu.VMEM((CH, DD), jnp.float32),
            pltpu.VMEM((ZR, DD), jnp.float32),
            pltpu.VMEM_SHARED((NN, DD), jnp.float32),
            pltpu.SemaphoreType.DMA,
        ],
    )
    def scat_kernel(src_hbm, dst_hbm, y_hbm, out_hbm, srcv, dstv, rowsv, zerov,
                    acc, sem):
        cid = lax.axis_index("c")
        sid = lax.axis_index("s")
        wid = sid * NC + cid

        @pl.loop(0, ZR)
        def _(i):
            for j in range(DD // 16):
                zerov[i, pl.ds(j * 16, 16)] = jnp.zeros((16,), jnp.float32)

        _zero_acc(zerov, acc, sid)
        plsc.subcore_barrier()

        @pl.loop(0, ITERS)
        def _(it):
            chunk = it * NW + wid

            @pl.when(chunk < NCHUNK)
            def _():
                base = chunk * CH
                pltpu.sync_copy(src_hbm.at[pl.ds(base, CH)], srcv)
                pltpu.sync_copy(dst_hbm.at[pl.ds(base, CH)], dstv)
                pltpu.async_copy(y_hbm.at[srcv], rowsv, sem).wait()
                pltpu.sync_copy(rowsv, acc.at[dstv], add=True)

        plsc.subcore_barrier()
        _copy_out(acc, out_hbm, cid, sid)

    return scat_kernel(src_arr, dst_arr, y)


def _deg_cols(degp_blk):
    deg = degp_blk[0, :, 0:1] + degp_blk[1, :, 0:1] + 1.0
    return lax.rsqrt(deg), 1.0 / deg


def _tc_prep(x, W1, b1, degp):
    def body(xr, wr, br, dpr, yr, sr):
        dis, dinv = _deg_cols(dpr[...])
        xw = jnp.dot(xr[...], wr[...], preferred_element_type=jnp.float32)
        yr[...] = dis * xw
        sr[...] = dinv * xw + br[...]

    return pl.pallas_call(
        body,
        grid=(NN // RB,),
        in_specs=[
            pl.BlockSpec((RB, DD), lambda i: (i, 0)),
            pl.BlockSpec((DD, DD), lambda i: (0, 0)),
            pl.BlockSpec((1, DD), lambda i: (0, 0)),
            pl.BlockSpec((NC, RB, DEGW), lambda i: (0, i, 0)),
        ],
        out_specs=[
            pl.BlockSpec((RB, DD), lambda i: (i, 0)),
            pl.BlockSpec((RB, DD), lambda i: (i, 0)),
        ],
        out_shape=[
            jax.ShapeDtypeStruct((NN, DD), jnp.float32),
            jax.ShapeDtypeStruct((NN, DD), jnp.float32),
        ],
    )(x, W1, b1, degp)


def _tc_mid(p, degp, self1, W2, b2):
    def body(pr, dpr, s1r, wr, br, yr, s2r):
        dis, dinv = _deg_cols(dpr[...])
        h1 = jnp.maximum(dis * (pr[0] + pr[1]) + s1r[...], 0.0)
        xw = jnp.dot(h1, wr[...], preferred_element_type=jnp.float32)
        yr[...] = dis * xw
        s2r[...] = dinv * xw + br[...]

    return pl.pallas_call(
        body,
        grid=(NN // RB,),
        in_specs=[
            pl.BlockSpec((NC, RB, DD), lambda i: (0, i, 0)),
            pl.BlockSpec((NC, RB, DEGW), lambda i: (0, i, 0)),
            pl.BlockSpec((RB, DD), lambda i: (i, 0)),
            pl.BlockSpec((DD, DD), lambda i: (0, 0)),
            pl.BlockSpec((1, DD), lambda i: (0, 0)),
        ],
        out_specs=[
            pl.BlockSpec((RB, DD), lambda i: (i, 0)),
            pl.BlockSpec((RB, DD), lambda i: (i, 0)),
        ],
        out_shape=[
            jax.ShapeDtypeStruct((NN, DD), jnp.float32),
            jax.ShapeDtypeStruct((NN, DD), jnp.float32),
        ],
    )(p, degp, self1, W2, b2)


def _tc_final(p, degp, self2, fc1_wt, fc1_b, fc2_wt, fc2_b):
    def body(pr, dpr, s2r, w1r, b1r, w2r, b2r, hr, zr):
        dis, _ = _deg_cols(dpr[...])
        h = dis * (pr[0] + pr[1]) + s2r[...]
        hr[...] = h
        z1 = jnp.maximum(
            jnp.dot(h, w1r[...], preferred_element_type=jnp.float32) + b1r[...], 0.0)
        zr[...] = jnp.dot(z1, w2r[...], preferred_element_type=jnp.float32) + b2r[...]

    return pl.pallas_call(
        body,
        grid=(NN // RB,),
        in_specs=[
            pl.BlockSpec((NC, RB, DD), lambda i: (0, i, 0)),
            pl.BlockSpec((NC, RB, DEGW), lambda i: (0, i, 0)),
            pl.BlockSpec((RB, DD), lambda i: (i, 0)),
            pl.BlockSpec((DD, DD), lambda i: (0, 0)),
            pl.BlockSpec((1, DD), lambda i: (0, 0)),
            pl.BlockSpec((DD, DD), lambda i: (0, 0)),
            pl.BlockSpec((1, DD), lambda i: (0, 0)),
        ],
        out_specs=[
            pl.BlockSpec((RB, DD), lambda i: (i, 0)),
            pl.BlockSpec((RB, DD), lambda i: (i, 0)),
        ],
        out_shape=[
            jax.ShapeDtypeStruct((NN, DD), jnp.float32),
            jax.ShapeDtypeStruct((NN, DD), jnp.float32),
        ],
    )(p, degp, self2, fc1_wt, fc1_b, fc2_wt, fc2_b)


def kernel(x, edge_index, W1, b1, W2, b2, fc1_w, fc1_b, fc2_w, fc2_b):
    src_arr = edge_index[0]
    dst_arr = edge_index[1]
    degp = _sc_degree(dst_arr)
    y1, self1 = _tc_prep(x, W1, b1.reshape(1, DD), degp)
    p1 = _sc_scatter(src_arr, dst_arr, y1)
    y2, self2 = _tc_mid(p1, degp, self1, W2, b2.reshape(1, DD))
    p2 = _sc_scatter(src_arr, dst_arr, y2)
    h, z = _tc_final(p2, degp, self2, fc1_w.T, fc1_b.reshape(1, DD),
                     fc2_w.T, fc2_b.reshape(1, DD))
    return (h, z)
